# Initial kernel scaffold; baseline (speedup 1.0000x reference)
import jax
import jax.numpy as jnp
from jax.experimental import pallas as pl


def kernel(node_token_ids, node_category_ids, edge_index, edge_category_ids, node_batch_offsets, images, emb_table, edge_super_W, edge_super_b, input_proj_W, input_proj_b, image_proj_W, image_proj_b, image_token, gat0_W, gat0_We, gat0_asrc, gat0_adst, gat0_aedge, gat0_b, gat1_W, gat1_We, gat1_asrc, gat1_adst, gat1_aedge, gat1_b):
    raise NotImplementedError("write your pallas kernel here")


# speedup vs baseline: 1.0904x; 1.0904x over previous
import functools

import jax
import jax.numpy as jnp
from jax.experimental import pallas as pl
from jax.experimental.pallas import tpu as pltpu

N = 50000
E = 800000
V = 100000
F = 128
B = 4
L = 12500
IMG = 512
H = 4
C = 64

VBLK = 1000


def _act(t):
    return jnp.where(t > 0, t, 0.01 * t)


def _precompute_body(emb_ref, wtop_ref, wbot_ref, wes_ref, bes_ref, q_ref,
                     q0_ref, acat_ref, atok_ref, tale_ref):
    e = emb_ref[...]
    t = _act(e)
    acat_ref[...] = jnp.dot(t, wtop_ref[...], preferred_element_type=jnp.float32)
    atok_ref[...] = jnp.dot(t, wbot_ref[...], preferred_element_type=jnp.float32)
    tef = jnp.dot(e, wes_ref[...], preferred_element_type=jnp.float32) + bes_ref[...]
    tale_ref[...] = (jnp.dot(_act(tef), q_ref[...],
                             preferred_element_type=jnp.float32) + q0_ref[...])


def _precompute_tables(emb_table, wtop, wbot, wes, bes, q, q0):
    grid = (V // VBLK,)
    full = lambda shape: pl.BlockSpec(shape, lambda i: (0, 0))
    return pl.pallas_call(
        _precompute_body,
        grid=grid,
        in_specs=[
            pl.BlockSpec((VBLK, C), lambda i: (i, 0)),
            full((C, C)),
            full((C, C)),
            full((C, F)),
            pl.BlockSpec((1, F), lambda i: (0, 0)),
            full((F, 8)),
            pl.BlockSpec((1, 8), lambda i: (0, 0)),
        ],
        out_specs=[
            pl.BlockSpec((VBLK, C), lambda i: (i, 0)),
            pl.BlockSpec((VBLK, C), lambda i: (i, 0)),
            pl.BlockSpec((VBLK, 8), lambda i: (i, 0)),
        ],
        out_shape=[
            jax.ShapeDtypeStruct((V, C), jnp.float32),
            jax.ShapeDtypeStruct((V, C), jnp.float32),
            jax.ShapeDtypeStruct((V, 8), jnp.float32),
        ],
    )(emb_table, wtop, wbot, wes, bes, q, q0)


def _fold(W, a, dh):
    return jnp.einsum('khd,hd->kh', W.reshape(W.shape[0], H, dh), a)


def kernel(node_token_ids, node_category_ids, edge_index, edge_category_ids,
           node_batch_offsets, images, emb_table, edge_super_W, edge_super_b,
           input_proj_W, input_proj_b, image_proj_W, image_proj_b, image_token,
           gat0_W, gat0_We, gat0_asrc, gat0_adst, gat0_aedge, gat0_b,
           gat1_W, gat1_We, gat1_asrc, gat1_adst, gat1_aedge, gat1_b):
    num_nodes = N + 1
    dh0 = C // H
    dh1 = F // H

    Ve0 = _fold(gat0_We, gat0_aedge, dh0)
    Ve1 = _fold(gat1_We, gat1_aedge, dh1)
    Vs0 = _fold(gat0_W, gat0_asrc, dh0)
    Vd0 = _fold(gat0_W, gat0_adst, dh0)
    Vs1 = _fold(gat1_W, gat1_asrc, dh1)
    Vd1 = _fold(gat1_W, gat1_adst, dh1)

    VeC = jnp.concatenate([Ve0, Ve1], axis=1)
    Q = input_proj_W @ VeC
    q0 = (input_proj_b @ VeC)[None]

    wtop = input_proj_W[:F // 2]
    wbot = input_proj_W[F // 2:]

    A_cat, A_tok, T_ale = _precompute_tables(
        emb_table, wtop, wbot, edge_super_W, edge_super_b[None], Q, q0)

    node_proj = A_cat[node_category_ids] + A_tok[node_token_ids] + input_proj_b
    node_proj = jnp.concatenate([node_proj, input_proj_b[None]], axis=0)

    src = edge_index[0]
    dst = edge_index[1]
    ale = T_ale[edge_category_ids]

    def layer(x, W, Vs, Vd, b, ale_l, dh):
        xw = x @ W
        als = x @ Vs
        ald = x @ Vd
        t = als[src] + ald[dst] + ale_l
        logits = jnp.where(t > 0, t, 0.2 * t)
        ex = jnp.exp(logits)
        den = jax.ops.segment_sum(ex, dst, num_segments=num_nodes)
        msg = xw[src].reshape(E, H, dh) * ex[..., None]
        S = jax.ops.segment_sum(msg, dst, num_segments=num_nodes)
        out = S / (den[..., None] + 1e-16)
        return out.reshape(num_nodes, H * dh) + b

    h = layer(node_proj, gat0_W, Vs0, Vd0, gat0_b, ale[:, :4], dh0)
    h = jax.nn.elu(h)
    h = layer(h, gat1_W, Vs1, Vd1, gat1_b, ale[:, 4:], dh1)

    img_tok = jnp.tile(image_token, (B, 1))
    concat = jnp.concatenate([img_tok, images], axis=1)
    image_rep = (concat @ image_proj_W + image_proj_b)[:, None, :]
    sequences = jnp.take(h, node_batch_offsets, axis=0)
    return jnp.concatenate([sequences, image_rep], axis=1)
